# baseline (speedup 1.0000x reference)
import jax
import jax.numpy as jnp
from jax import lax
from jax.experimental import pallas as pl
from jax.experimental.pallas import tpu as pltpu

T_IN = 553
PAD = 6
KW = 12
CIN = 4
C1 = 22
C2 = 20
T_OUT = T_IN + 2 * PAD - KW + 1
POOL_W, POOL_S = 62, 12
N_POOL = (T_OUT - POOL_W) // POOL_S + 1
NCLS = 4
EPS = 1e-5

T_LANE = 640
T_SRC = 656
C2P = 24
QP = 128
KIM = KW * CIN
B_BLK = 128
P_BLK = B_BLK // 2
CH_P = 64
KIM2 = 2 * KIM
C2P2 = 2 * C2P


def _fold_params(p):
    hp = lax.Precision.HIGHEST
    a1 = p['bn1_gamma'] / jnp.sqrt(p['bn1_var'] + EPS)
    c1 = p['bn1_beta'] + a1 * (p['conv1_b'] - p['bn1_mean'])
    a2 = p['bn2_gamma'] / jnp.sqrt(p['bn2_var'] + EPS)
    c2 = p['bn2_beta'] + a2 * (p['conv2_b'] - p['bn2_mean'])

    w1 = p['conv1_w'][:, 0, :, 0]
    w2 = p['conv2_w'][:, :, 0, :]

    wf = jnp.einsum('ock,ci->oki', w2 * a1[None, :, None], w1, precision=hp)
    wf = (a2[:, None, None] * wf).reshape(C2, KIM)
    wf = jnp.pad(wf, ((0, C2P - C2), (0, 0)))

    s_idx = jnp.arange(T_SRC)
    mask = ((s_idx >= PAD) & (s_idx < PAD + T_IN)).astype(jnp.float32)
    mask_sh = jnp.stack([mask[k:k + T_LANE] for k in range(KW)], 0)
    cvec = jnp.sum(w2 * c1[None, :, None], axis=1)
    bias = (a2[:, None] * jnp.einsum('ok,kt->ot', cvec, mask_sh, precision=hp)
            + c2[:, None])
    bias = jnp.pad(bias, ((0, C2P - C2), (0, 0)))

    tt = jnp.arange(T_LANE)[:, None]
    qq = jnp.arange(QP)[None, :]
    pmat = ((tt >= POOL_S * qq) & (tt < POOL_S * qq + POOL_W)
            & (tt < T_OUT) & (qq < N_POOL)).astype(jnp.float32) / POOL_W

    oo = jnp.arange(C2P)[:, None]
    padfix = ((oo >= C2) | (qq >= N_POOL)).astype(jnp.float32)

    wc = p['cls_w'].reshape(NCLS, C2, N_POOL)
    wc = jnp.pad(wc, ((0, 0), (0, C2P - C2), (0, QP - N_POOL)))

    wc2 = wc.transpose(2, 0, 1).reshape(QP, NCLS * C2P)
    m24 = jnp.concatenate([jnp.eye(C2P, dtype=jnp.float32)] * NCLS, axis=1)
    mtile = jnp.concatenate([m24] * (2 * CH_P), axis=0)
    gmat = jnp.kron(jnp.eye(NCLS, dtype=jnp.float32),
                    jnp.ones((C2P, 1), jnp.float32))

    bc = jnp.pad(p['cls_b'][None, :], ((0, 0), (0, QP - NCLS)))

    wf_r = wf.reshape(C2P, KW, CIN)
    w2a = jnp.pad(wf_r, ((0, 0), (0, 0), (0, CIN))).reshape(C2P, KIM2)
    w2b = jnp.pad(wf_r, ((0, 0), (0, 0), (CIN, 0))).reshape(C2P, KIM2)
    wpair = jnp.concatenate([w2a, w2b], axis=0)
    bias2 = jnp.concatenate([bias, bias], axis=0)
    return wpair, bias2, pmat, padfix, wc2, mtile, gmat, bc


def _kernel_body(x_ref, wf_ref, bias_ref, pmat_ref, padfix_ref,
                 wc2_ref, mtile_ref, gmat_ref, bc_ref, out_ref,
                 xp_s, xi_s, sq_s):
    xp_s[:, :, 0:PAD] = jnp.zeros((P_BLK, 2 * CIN, PAD), jnp.float32)
    xp_s[:, 0:CIN, PAD:PAD + T_IN] = x_ref[0:P_BLK, 0]
    xp_s[:, CIN:2 * CIN, PAD:PAD + T_IN] = x_ref[P_BLK:2 * P_BLK, 0]
    xp_s[:, :, PAD + T_IN:T_SRC] = jnp.zeros(
        (P_BLK, 2 * CIN, T_SRC - PAD - T_IN), jnp.float32)

    wf = wf_ref[...]
    bias = bias_ref[...]
    out_ref[...] = jnp.zeros_like(out_ref)

    for c in range(P_BLK // CH_P):
        p0 = c * CH_P
        for k in range(KW):
            xi_s[p0:p0 + CH_P, 2 * CIN * k:2 * CIN * (k + 1), :] = \
                xp_s[p0:p0 + CH_P, :, k:k + T_LANE]

        for b in range(p0, p0 + CH_P):
            z = jnp.dot(wf, xi_s[b],
                        preferred_element_type=jnp.float32) + bias
            sq_s[b] = z * z

        sq_flat = sq_s[p0:p0 + CH_P].reshape(CH_P * C2P2, T_LANE)
        pooled = jnp.dot(sq_flat, pmat_ref[...],
                         preferred_element_type=jnp.float32)

        padfix3 = jnp.broadcast_to(padfix_ref[...], (2 * CH_P, C2P, QP))
        logp = jnp.log(pooled + padfix3.reshape(CH_P * C2P2, QP))

        u = jnp.dot(logp, wc2_ref[...],
                    preferred_element_type=jnp.float32)
        um = u * mtile_ref[...]
        w2 = jnp.dot(um, gmat_ref[...],
                     preferred_element_type=jnp.float32)
        s = jnp.sum(w2.reshape(2 * CH_P, C2P, NCLS), axis=1)
        out_ref[2 * p0:2 * (p0 + CH_P), 0:NCLS] = s + bc_ref[:, 0:NCLS]


def kernel(x, conv1_w, conv1_b, bn1_gamma, bn1_beta, bn1_mean, bn1_var,
           conv2_w, conv2_b, bn2_gamma, bn2_beta, bn2_mean, bn2_var,
           cls_w, cls_b):
    p = dict(
        conv1_w=conv1_w, conv1_b=conv1_b,
        bn1_gamma=bn1_gamma, bn1_beta=bn1_beta,
        bn1_mean=bn1_mean, bn1_var=bn1_var,
        conv2_w=conv2_w, conv2_b=conv2_b,
        bn2_gamma=bn2_gamma, bn2_beta=bn2_beta,
        bn2_mean=bn2_mean, bn2_var=bn2_var,
        cls_w=cls_w, cls_b=cls_b,
    )
    wf, bias, pmat, padfix, wc2, mtile, gmat, bc = _fold_params(p)

    B = x.shape[0]

    out = pl.pallas_call(
        _kernel_body,
        out_shape=jax.ShapeDtypeStruct((B, QP), jnp.float32),
        grid=(B // B_BLK,),
        in_specs=[
            pl.BlockSpec((B_BLK, 1, CIN, T_IN), lambda g: (g, 0, 0, 0)),
            pl.BlockSpec((C2P2, KIM2), lambda g: (0, 0)),
            pl.BlockSpec((C2P2, T_LANE), lambda g: (0, 0)),
            pl.BlockSpec((T_LANE, QP), lambda g: (0, 0)),
            pl.BlockSpec((C2P, QP), lambda g: (0, 0)),
            pl.BlockSpec((QP, NCLS * C2P), lambda g: (0, 0)),
            pl.BlockSpec((2 * CH_P * C2P, NCLS * C2P), lambda g: (0, 0)),
            pl.BlockSpec((NCLS * C2P, NCLS), lambda g: (0, 0)),
            pl.BlockSpec((1, QP), lambda g: (0, 0)),
        ],
        out_specs=pl.BlockSpec((B_BLK, QP), lambda g: (g, 0)),
        scratch_shapes=[
            pltpu.VMEM((P_BLK, 2 * CIN, T_SRC), jnp.float32),
            pltpu.VMEM((P_BLK, KIM2, T_LANE), jnp.float32),
            pltpu.VMEM((P_BLK, C2P2, T_LANE), jnp.float32),
        ],
        compiler_params=pltpu.CompilerParams(
            dimension_semantics=("parallel",),
            vmem_limit_bytes=64 * 1024 * 1024),
    )(x, wf, bias, pmat, padfix, wc2, mtile, gmat, bc)
    outs = out[:, :NCLS].reshape(B // B_BLK, P_BLK, 2, NCLS)
    return outs.swapaxes(1, 2).reshape(B, NCLS)


# speedup vs baseline: 1.0001x; 1.0001x over previous
import jax
import jax.numpy as jnp
from jax import lax
from jax.experimental import pallas as pl
from jax.experimental.pallas import tpu as pltpu

T_IN = 553
PAD = 6
KW = 12
CIN = 4
C1 = 22
C2 = 20
T_OUT = T_IN + 2 * PAD - KW + 1
POOL_W, POOL_S = 62, 12
N_POOL = (T_OUT - POOL_W) // POOL_S + 1
NCLS = 4
EPS = 1e-5

T_LANE = 640
T_SRC = 656
C2P = 24
QP = 128
KIM = KW * CIN
B_BLK = 128
P_BLK = B_BLK // 2
CH_P = 64
KIM2 = 2 * KIM
C2P2 = 2 * C2P


def _fold_params(p):
    hp = lax.Precision.HIGHEST
    a1 = p['bn1_gamma'] / jnp.sqrt(p['bn1_var'] + EPS)
    c1 = p['bn1_beta'] + a1 * (p['conv1_b'] - p['bn1_mean'])
    a2 = p['bn2_gamma'] / jnp.sqrt(p['bn2_var'] + EPS)
    c2 = p['bn2_beta'] + a2 * (p['conv2_b'] - p['bn2_mean'])

    w1 = p['conv1_w'][:, 0, :, 0]
    w2 = p['conv2_w'][:, :, 0, :]

    wf = jnp.einsum('ock,ci->oki', w2 * a1[None, :, None], w1, precision=hp)
    wf = (a2[:, None, None] * wf).reshape(C2, KIM)
    wf = jnp.pad(wf, ((0, C2P - C2), (0, 0)))

    s_idx = jnp.arange(T_SRC)
    mask = ((s_idx >= PAD) & (s_idx < PAD + T_IN)).astype(jnp.float32)
    mask_sh = jnp.stack([mask[k:k + T_LANE] for k in range(KW)], 0)
    cvec = jnp.sum(w2 * c1[None, :, None], axis=1)
    bias = (a2[:, None] * jnp.einsum('ok,kt->ot', cvec, mask_sh, precision=hp)
            + c2[:, None])
    bias = jnp.pad(bias, ((0, C2P - C2), (0, 0)))

    tt = jnp.arange(T_LANE)[:, None]
    qq = jnp.arange(QP)[None, :]
    pmat = ((tt >= POOL_S * qq) & (tt < POOL_S * qq + POOL_W)
            & (tt < T_OUT) & (qq < N_POOL)).astype(jnp.float32) / POOL_W

    oo = jnp.arange(C2P)[:, None]
    padfix = ((oo >= C2) | (qq >= N_POOL)).astype(jnp.float32)

    wc = p['cls_w'].reshape(NCLS, C2, N_POOL)
    wc = jnp.pad(wc, ((0, 0), (0, C2P - C2), (0, QP - N_POOL)))

    wc2 = wc.transpose(2, 0, 1).reshape(QP, NCLS * C2P)
    m24 = jnp.concatenate([jnp.eye(C2P, dtype=jnp.float32)] * NCLS, axis=1)
    mtile = jnp.concatenate([m24] * (2 * CH_P), axis=0)
    gmat = jnp.kron(jnp.eye(NCLS, dtype=jnp.float32),
                    jnp.ones((C2P, 1), jnp.float32))

    bc = jnp.pad(p['cls_b'][None, :], ((0, 0), (0, QP - NCLS)))

    wf_r = wf.reshape(C2P, KW, CIN)
    w2a = jnp.pad(wf_r, ((0, 0), (0, 0), (0, CIN))).reshape(C2P, KIM2)
    w2b = jnp.pad(wf_r, ((0, 0), (0, 0), (CIN, 0))).reshape(C2P, KIM2)
    wpair = jnp.concatenate([w2a, w2b], axis=0)
    bias2 = jnp.concatenate([bias, bias], axis=0)
    return wpair, bias2, pmat, padfix, wc2, mtile, gmat, bc


def _kernel_body(x_ref, wf_ref, bias_ref, pmat_ref, padfix_ref,
                 wc2_ref, mtile_ref, gmat_ref, bc_ref, out_ref,
                 xp_s, xi_s, sq_s):
    xp_s[:, :, 0:PAD] = jnp.zeros((P_BLK, 2 * CIN, PAD), jnp.float32)
    x3 = x_ref[...].reshape(B_BLK, CIN, T_IN)
    xp_s[:, 0:CIN, PAD:PAD + T_IN] = x3[0:P_BLK]
    xp_s[:, CIN:2 * CIN, PAD:PAD + T_IN] = x3[P_BLK:2 * P_BLK]
    xp_s[:, :, PAD + T_IN:T_SRC] = jnp.zeros(
        (P_BLK, 2 * CIN, T_SRC - PAD - T_IN), jnp.float32)

    wf = wf_ref[...]
    bias = bias_ref[...]
    out_ref[...] = jnp.zeros_like(out_ref)

    for c in range(P_BLK // CH_P):
        p0 = c * CH_P
        for k in range(KW):
            xi_s[p0:p0 + CH_P, 2 * CIN * k:2 * CIN * (k + 1), :] = \
                xp_s[p0:p0 + CH_P, :, k:k + T_LANE]

        for b in range(p0, p0 + CH_P):
            z = jnp.dot(wf, xi_s[b],
                        preferred_element_type=jnp.float32) + bias
            sq_s[b] = z * z

        sq_flat = sq_s[p0:p0 + CH_P].reshape(CH_P * C2P2, T_LANE)
        pooled = jnp.dot(sq_flat, pmat_ref[...],
                         preferred_element_type=jnp.float32)

        padfix3 = jnp.broadcast_to(padfix_ref[...], (2 * CH_P, C2P, QP))
        logp = jnp.log(pooled + padfix3.reshape(CH_P * C2P2, QP))

        u = jnp.dot(logp, wc2_ref[...],
                    preferred_element_type=jnp.float32)
        um = u * mtile_ref[...]
        w2 = jnp.dot(um, gmat_ref[...],
                     preferred_element_type=jnp.float32)
        s = jnp.sum(w2.reshape(2 * CH_P, C2P, NCLS), axis=1)
        out_ref[2 * p0:2 * (p0 + CH_P), 0:NCLS] = s + bc_ref[:, 0:NCLS]


def kernel(x, conv1_w, conv1_b, bn1_gamma, bn1_beta, bn1_mean, bn1_var,
           conv2_w, conv2_b, bn2_gamma, bn2_beta, bn2_mean, bn2_var,
           cls_w, cls_b):
    p = dict(
        conv1_w=conv1_w, conv1_b=conv1_b,
        bn1_gamma=bn1_gamma, bn1_beta=bn1_beta,
        bn1_mean=bn1_mean, bn1_var=bn1_var,
        conv2_w=conv2_w, conv2_b=conv2_b,
        bn2_gamma=bn2_gamma, bn2_beta=bn2_beta,
        bn2_mean=bn2_mean, bn2_var=bn2_var,
        cls_w=cls_w, cls_b=cls_b,
    )
    wf, bias, pmat, padfix, wc2, mtile, gmat, bc = _fold_params(p)

    B = x.shape[0]

    out = pl.pallas_call(
        _kernel_body,
        out_shape=jax.ShapeDtypeStruct((B, QP), jnp.float32),
        grid=(B // B_BLK,),
        in_specs=[
            pl.BlockSpec((B_BLK, 1, CIN, T_IN), lambda g: (g, 0, 0, 0)),
            pl.BlockSpec((C2P2, KIM2), lambda g: (0, 0)),
            pl.BlockSpec((C2P2, T_LANE), lambda g: (0, 0)),
            pl.BlockSpec((T_LANE, QP), lambda g: (0, 0)),
            pl.BlockSpec((C2P, QP), lambda g: (0, 0)),
            pl.BlockSpec((QP, NCLS * C2P), lambda g: (0, 0)),
            pl.BlockSpec((2 * CH_P * C2P, NCLS * C2P), lambda g: (0, 0)),
            pl.BlockSpec((NCLS * C2P, NCLS), lambda g: (0, 0)),
            pl.BlockSpec((1, QP), lambda g: (0, 0)),
        ],
        out_specs=pl.BlockSpec((B_BLK, QP), lambda g: (g, 0)),
        scratch_shapes=[
            pltpu.VMEM((P_BLK, 2 * CIN, T_SRC), jnp.float32),
            pltpu.VMEM((P_BLK, KIM2, T_LANE), jnp.float32),
            pltpu.VMEM((P_BLK, C2P2, T_LANE), jnp.float32),
        ],
        compiler_params=pltpu.CompilerParams(
            dimension_semantics=("parallel",),
            vmem_limit_bytes=64 * 1024 * 1024),
    )(x, wf, bias, pmat, padfix, wc2, mtile, gmat, bc)
    outs = out[:, :NCLS].reshape(B // B_BLK, P_BLK, 2, NCLS)
    return outs.swapaxes(1, 2).reshape(B, NCLS)


# speedup vs baseline: 1.1287x; 1.1286x over previous
import jax
import jax.numpy as jnp
from jax import lax
from jax.experimental import pallas as pl
from jax.experimental.pallas import tpu as pltpu

T_IN = 553
PAD = 6
KW = 12
CIN = 4
C1 = 22
C2 = 20
T_OUT = T_IN + 2 * PAD - KW + 1
POOL_W, POOL_S = 62, 12
N_POOL = (T_OUT - POOL_W) // POOL_S + 1
NCLS = 4
EPS = 1e-5

T_LANE = 640
T_SRC = 656
C2P = 24
QP = 128
KIM = KW * CIN
B_BLK = 128
P_BLK = B_BLK // 2
CH_P = 64
KIM2 = 2 * KIM
C2P2 = 2 * C2P


def _fold_params(p):
    hp = lax.Precision.HIGHEST
    a1 = p['bn1_gamma'] / jnp.sqrt(p['bn1_var'] + EPS)
    c1 = p['bn1_beta'] + a1 * (p['conv1_b'] - p['bn1_mean'])
    a2 = p['bn2_gamma'] / jnp.sqrt(p['bn2_var'] + EPS)
    c2 = p['bn2_beta'] + a2 * (p['conv2_b'] - p['bn2_mean'])

    w1 = p['conv1_w'][:, 0, :, 0]
    w2 = p['conv2_w'][:, :, 0, :]

    wf = jnp.einsum('ock,ci->oki', w2 * a1[None, :, None], w1, precision=hp)
    wf = (a2[:, None, None] * wf).reshape(C2, KIM)
    wf = jnp.pad(wf, ((0, C2P - C2), (0, 0)))

    s_idx = jnp.arange(T_SRC)
    mask = ((s_idx >= PAD) & (s_idx < PAD + T_IN)).astype(jnp.float32)
    mask_sh = jnp.stack([mask[k:k + T_LANE] for k in range(KW)], 0)
    cvec = jnp.sum(w2 * c1[None, :, None], axis=1)
    bias = (a2[:, None] * jnp.einsum('ok,kt->ot', cvec, mask_sh, precision=hp)
            + c2[:, None])
    bias = jnp.pad(bias, ((0, C2P - C2), (0, 0)))

    tt = jnp.arange(T_LANE)[:, None]
    qq = jnp.arange(QP)[None, :]
    pmat = ((tt >= POOL_S * qq) & (tt < POOL_S * qq + POOL_W)
            & (tt < T_OUT) & (qq < N_POOL)).astype(jnp.float32) / POOL_W

    oo = jnp.arange(C2P)[:, None]
    padfix = ((oo >= C2) | (qq >= N_POOL)).astype(jnp.float32)

    wc = p['cls_w'].reshape(NCLS, C2, N_POOL)
    wc = jnp.pad(wc, ((0, 0), (0, C2P - C2), (0, QP - N_POOL)))

    wc2 = wc.transpose(2, 0, 1).reshape(QP, NCLS * C2P)
    m24 = jnp.concatenate([jnp.eye(C2P, dtype=jnp.float32)] * NCLS, axis=1)
    mtile = jnp.concatenate([m24] * (2 * CH_P), axis=0)
    gmat = jnp.kron(jnp.eye(NCLS, dtype=jnp.float32),
                    jnp.ones((C2P, 1), jnp.float32))

    bc = jnp.pad(p['cls_b'][None, :], ((0, 0), (0, QP - NCLS)))

    wf_r = wf.reshape(C2P, KW, CIN)
    w2a = jnp.pad(wf_r, ((0, 0), (0, 0), (0, CIN))).reshape(C2P, KIM2)
    w2b = jnp.pad(wf_r, ((0, 0), (0, 0), (CIN, 0))).reshape(C2P, KIM2)
    wpair = jnp.concatenate([w2a, w2b], axis=0)
    bias2 = jnp.concatenate([bias, bias], axis=0)
    return wpair, bias2, pmat, padfix, wc2, mtile, gmat, bc


def _kernel_body(x_ref, wf_ref, bias_ref, pmat_ref, padfix_ref,
                 wc2_ref, mtile_ref, gmat_ref, bc_ref, out_ref,
                 xp_s, xi_s, sq_s):
    xp_s[:, :, 0:PAD] = jnp.zeros((P_BLK, 2 * CIN, PAD), jnp.float32)
    xp_s[:, 0:CIN, PAD:PAD + T_IN] = x_ref[0:P_BLK]
    xp_s[:, CIN:2 * CIN, PAD:PAD + T_IN] = x_ref[P_BLK:2 * P_BLK]
    xp_s[:, :, PAD + T_IN:T_SRC] = jnp.zeros(
        (P_BLK, 2 * CIN, T_SRC - PAD - T_IN), jnp.float32)

    wf = wf_ref[...]
    bias = bias_ref[...]
    out_ref[...] = jnp.zeros_like(out_ref)

    for c in range(P_BLK // CH_P):
        p0 = c * CH_P
        for k in range(KW):
            xi_s[p0:p0 + CH_P, 2 * CIN * k:2 * CIN * (k + 1), :] = \
                xp_s[p0:p0 + CH_P, :, k:k + T_LANE]

        for b in range(p0, p0 + CH_P):
            z = jnp.dot(wf, xi_s[b],
                        preferred_element_type=jnp.float32) + bias
            sq_s[b] = z * z

        sq_flat = sq_s[p0:p0 + CH_P].reshape(CH_P * C2P2, T_LANE)
        pooled = jnp.dot(sq_flat, pmat_ref[...],
                         preferred_element_type=jnp.float32)

        padfix3 = jnp.broadcast_to(padfix_ref[...], (2 * CH_P, C2P, QP))
        logp = jnp.log(pooled + padfix3.reshape(CH_P * C2P2, QP))

        u = jnp.dot(logp, wc2_ref[...],
                    preferred_element_type=jnp.float32)
        um = u * mtile_ref[...]
        w2 = jnp.dot(um, gmat_ref[...],
                     preferred_element_type=jnp.float32)
        s = jnp.sum(w2.reshape(2 * CH_P, C2P, NCLS), axis=1)
        out_ref[2 * p0:2 * (p0 + CH_P), 0:NCLS] = s + bc_ref[:, 0:NCLS]


def kernel(x, conv1_w, conv1_b, bn1_gamma, bn1_beta, bn1_mean, bn1_var,
           conv2_w, conv2_b, bn2_gamma, bn2_beta, bn2_mean, bn2_var,
           cls_w, cls_b):
    p = dict(
        conv1_w=conv1_w, conv1_b=conv1_b,
        bn1_gamma=bn1_gamma, bn1_beta=bn1_beta,
        bn1_mean=bn1_mean, bn1_var=bn1_var,
        conv2_w=conv2_w, conv2_b=conv2_b,
        bn2_gamma=bn2_gamma, bn2_beta=bn2_beta,
        bn2_mean=bn2_mean, bn2_var=bn2_var,
        cls_w=cls_w, cls_b=cls_b,
    )
    wf, bias, pmat, padfix, wc2, mtile, gmat, bc = _fold_params(p)

    B = x.shape[0]
    x2 = x[:, 0, :, :]

    out = pl.pallas_call(
        _kernel_body,
        out_shape=jax.ShapeDtypeStruct((B, QP), jnp.float32),
        grid=(B // B_BLK,),
        in_specs=[
            pl.BlockSpec((B_BLK, CIN, T_IN), lambda g: (g, 0, 0)),
            pl.BlockSpec((C2P2, KIM2), lambda g: (0, 0)),
            pl.BlockSpec((C2P2, T_LANE), lambda g: (0, 0)),
            pl.BlockSpec((T_LANE, QP), lambda g: (0, 0)),
            pl.BlockSpec((C2P, QP), lambda g: (0, 0)),
            pl.BlockSpec((QP, NCLS * C2P), lambda g: (0, 0)),
            pl.BlockSpec((2 * CH_P * C2P, NCLS * C2P), lambda g: (0, 0)),
            pl.BlockSpec((NCLS * C2P, NCLS), lambda g: (0, 0)),
            pl.BlockSpec((1, QP), lambda g: (0, 0)),
        ],
        out_specs=pl.BlockSpec((B_BLK, QP), lambda g: (g, 0)),
        scratch_shapes=[
            pltpu.VMEM((P_BLK, 2 * CIN, T_SRC), jnp.float32),
            pltpu.VMEM((P_BLK, KIM2, T_LANE), jnp.float32),
            pltpu.VMEM((P_BLK, C2P2, T_LANE), jnp.float32),
        ],
        compiler_params=pltpu.CompilerParams(
            dimension_semantics=("parallel",),
            vmem_limit_bytes=64 * 1024 * 1024),
    )(x2, wf, bias, pmat, padfix, wc2, mtile, gmat, bc)
    outs = out[:, :NCLS].reshape(B // B_BLK, P_BLK, 2, NCLS)
    return outs.swapaxes(1, 2).reshape(B, NCLS)


# speedup vs baseline: 1.1858x; 1.0506x over previous
import jax
import jax.numpy as jnp
from jax import lax
from jax.experimental import pallas as pl
from jax.experimental.pallas import tpu as pltpu

T_IN = 553
PAD = 6
KW = 12
CIN = 4
C1 = 22
C2 = 20
T_OUT = T_IN + 2 * PAD - KW + 1
POOL_W, POOL_S = 62, 12
N_POOL = (T_OUT - POOL_W) // POOL_S + 1
NCLS = 4
EPS = 1e-5

T_LANE = 640
T_SRC = 656
C2P = 24
QP = 128
KIM = KW * CIN
B_BLK = 128
P_BLK = B_BLK // 2
CH_P = 64
KIM2 = 2 * KIM
C2P2 = 2 * C2P


def _fold_params(p):
    hp = lax.Precision.HIGHEST
    a1 = p['bn1_gamma'] / jnp.sqrt(p['bn1_var'] + EPS)
    c1 = p['bn1_beta'] + a1 * (p['conv1_b'] - p['bn1_mean'])
    a2 = p['bn2_gamma'] / jnp.sqrt(p['bn2_var'] + EPS)
    c2 = p['bn2_beta'] + a2 * (p['conv2_b'] - p['bn2_mean'])

    w1 = p['conv1_w'][:, 0, :, 0]
    w2 = p['conv2_w'][:, :, 0, :]

    wf = jnp.einsum('ock,ci->oki', w2 * a1[None, :, None], w1, precision=hp)
    wf = (a2[:, None, None] * wf).reshape(C2, KIM)
    wf = jnp.pad(wf, ((0, C2P - C2), (0, 0)))

    s_idx = jnp.arange(T_SRC)
    mask = ((s_idx >= PAD) & (s_idx < PAD + T_IN)).astype(jnp.float32)
    mask_sh = jnp.stack([mask[k:k + T_LANE] for k in range(KW)], 0)
    cvec = jnp.sum(w2 * c1[None, :, None], axis=1)
    bias = (a2[:, None] * jnp.einsum('ok,kt->ot', cvec, mask_sh, precision=hp)
            + c2[:, None])
    bias = jnp.pad(bias, ((0, C2P - C2), (0, 0)))

    tt = jnp.arange(T_LANE)[:, None]
    qq = jnp.arange(QP)[None, :]
    pmat = ((tt >= POOL_S * qq) & (tt < POOL_S * qq + POOL_W)
            & (tt < T_OUT) & (qq < N_POOL)).astype(jnp.float32) / POOL_W

    oo = jnp.arange(C2P)[:, None]
    padfix = ((oo >= C2) | (qq >= N_POOL)).astype(jnp.float32)

    wc = p['cls_w'].reshape(NCLS, C2, N_POOL)
    wc = jnp.pad(wc, ((0, 0), (0, C2P - C2), (0, QP - N_POOL)))

    wc2 = wc.transpose(2, 0, 1).reshape(QP, NCLS * C2P)
    m24 = jnp.concatenate([jnp.eye(C2P, dtype=jnp.float32)] * NCLS, axis=1)
    mtile = jnp.concatenate([m24] * (2 * CH_P), axis=0)
    gmat = jnp.kron(jnp.eye(NCLS, dtype=jnp.float32),
                    jnp.ones((C2P, 1), jnp.float32))

    bc = jnp.pad(p['cls_b'][None, :], ((0, 0), (0, QP - NCLS)))

    wf_r = wf.reshape(C2P, KW, CIN)
    w2a = jnp.pad(wf_r, ((0, 0), (0, 0), (0, CIN))).reshape(C2P, KIM2)
    w2b = jnp.pad(wf_r, ((0, 0), (0, 0), (CIN, 0))).reshape(C2P, KIM2)
    wpair = jnp.concatenate([w2a, w2b], axis=0)
    bias2 = jnp.concatenate([bias, bias], axis=0)
    return wpair, bias2, pmat, padfix, wc2, mtile, gmat, bc


def _kernel_body(x_ref, wf_ref, bias_ref, pmat_ref, padfix_ref,
                 wc2_ref, mtile_ref, gmat_ref, bc_ref, out_ref,
                 xp_s, xi_s, sq_s):
    xp_s[:, :, 0:PAD] = jnp.zeros((P_BLK, 2 * CIN, PAD), jnp.float32)
    xcat = jnp.concatenate([x_ref[0:P_BLK], x_ref[P_BLK:2 * P_BLK]], axis=1)
    xp_s[:, :, PAD:PAD + T_IN] = xcat
    xp_s[:, :, PAD + T_IN:T_SRC] = jnp.zeros(
        (P_BLK, 2 * CIN, T_SRC - PAD - T_IN), jnp.float32)

    wf = wf_ref[...]
    bias = bias_ref[...]
    out_ref[...] = jnp.zeros_like(out_ref)

    for c in range(P_BLK // CH_P):
        p0 = c * CH_P
        for k in range(KW):
            xi_s[p0:p0 + CH_P, 2 * CIN * k:2 * CIN * (k + 1), :] = \
                xp_s[p0:p0 + CH_P, :, k:k + T_LANE]

        for b in range(p0, p0 + CH_P):
            z = jnp.dot(wf, xi_s[b],
                        preferred_element_type=jnp.float32) + bias
            sq_s[b] = z * z

        sq_flat = sq_s[p0:p0 + CH_P].reshape(CH_P * C2P2, T_LANE)
        pooled = jnp.dot(sq_flat, pmat_ref[...],
                         preferred_element_type=jnp.float32)

        padfix3 = jnp.broadcast_to(padfix_ref[...], (2 * CH_P, C2P, QP))
        logp = jnp.log(pooled + padfix3.reshape(CH_P * C2P2, QP))

        u = jnp.dot(logp, wc2_ref[...],
                    preferred_element_type=jnp.float32)
        um = u * mtile_ref[...]
        w2 = jnp.dot(um, gmat_ref[...],
                     preferred_element_type=jnp.float32)
        s = jnp.sum(w2.reshape(2 * CH_P, C2P, NCLS), axis=1)
        out_ref[2 * p0:2 * (p0 + CH_P), 0:NCLS] = s + bc_ref[:, 0:NCLS]


def kernel(x, conv1_w, conv1_b, bn1_gamma, bn1_beta, bn1_mean, bn1_var,
           conv2_w, conv2_b, bn2_gamma, bn2_beta, bn2_mean, bn2_var,
           cls_w, cls_b):
    p = dict(
        conv1_w=conv1_w, conv1_b=conv1_b,
        bn1_gamma=bn1_gamma, bn1_beta=bn1_beta,
        bn1_mean=bn1_mean, bn1_var=bn1_var,
        conv2_w=conv2_w, conv2_b=conv2_b,
        bn2_gamma=bn2_gamma, bn2_beta=bn2_beta,
        bn2_mean=bn2_mean, bn2_var=bn2_var,
        cls_w=cls_w, cls_b=cls_b,
    )
    wf, bias, pmat, padfix, wc2, mtile, gmat, bc = _fold_params(p)

    B = x.shape[0]
    x2 = x[:, 0, :, :]

    out = pl.pallas_call(
        _kernel_body,
        out_shape=jax.ShapeDtypeStruct((B, QP), jnp.float32),
        grid=(B // B_BLK,),
        in_specs=[
            pl.BlockSpec((B_BLK, CIN, T_IN), lambda g: (g, 0, 0)),
            pl.BlockSpec((C2P2, KIM2), lambda g: (0, 0)),
            pl.BlockSpec((C2P2, T_LANE), lambda g: (0, 0)),
            pl.BlockSpec((T_LANE, QP), lambda g: (0, 0)),
            pl.BlockSpec((C2P, QP), lambda g: (0, 0)),
            pl.BlockSpec((QP, NCLS * C2P), lambda g: (0, 0)),
            pl.BlockSpec((2 * CH_P * C2P, NCLS * C2P), lambda g: (0, 0)),
            pl.BlockSpec((NCLS * C2P, NCLS), lambda g: (0, 0)),
            pl.BlockSpec((1, QP), lambda g: (0, 0)),
        ],
        out_specs=pl.BlockSpec((B_BLK, QP), lambda g: (g, 0)),
        scratch_shapes=[
            pltpu.VMEM((P_BLK, 2 * CIN, T_SRC), jnp.float32),
            pltpu.VMEM((P_BLK, KIM2, T_LANE), jnp.float32),
            pltpu.VMEM((P_BLK, C2P2, T_LANE), jnp.float32),
        ],
        compiler_params=pltpu.CompilerParams(
            dimension_semantics=("parallel",),
            vmem_limit_bytes=64 * 1024 * 1024),
    )(x2, wf, bias, pmat, padfix, wc2, mtile, gmat, bc)
    outs = out[:, :NCLS].reshape(B // B_BLK, P_BLK, 2, NCLS)
    return outs.swapaxes(1, 2).reshape(B, NCLS)


# speedup vs baseline: 1.1987x; 1.0109x over previous
import jax
import jax.numpy as jnp
from jax import lax
from jax.experimental import pallas as pl
from jax.experimental.pallas import tpu as pltpu

T_IN = 553
PAD = 6
KW = 12
CIN = 4
C1 = 22
C2 = 20
T_OUT = T_IN + 2 * PAD - KW + 1
POOL_W, POOL_S = 62, 12
N_POOL = (T_OUT - POOL_W) // POOL_S + 1
NCLS = 4
EPS = 1e-5

T_LANE = 640
T_SRC = 656
C2P = 24
QP = 128
KIM = KW * CIN
B_BLK = 128
P_BLK = B_BLK // 2
CH_P = 64
KIM2 = 2 * KIM
C2P2 = 2 * C2P


def _fold_params(p):
    hp = lax.Precision.HIGHEST
    a1 = p['bn1_gamma'] / jnp.sqrt(p['bn1_var'] + EPS)
    c1 = p['bn1_beta'] + a1 * (p['conv1_b'] - p['bn1_mean'])
    a2 = p['bn2_gamma'] / jnp.sqrt(p['bn2_var'] + EPS)
    c2 = p['bn2_beta'] + a2 * (p['conv2_b'] - p['bn2_mean'])

    w1 = p['conv1_w'][:, 0, :, 0]
    w2 = p['conv2_w'][:, :, 0, :]

    wf = jnp.einsum('ock,ci->oki', w2 * a1[None, :, None], w1, precision=hp)
    wf = (a2[:, None, None] * wf).reshape(C2, KIM)
    wf = jnp.pad(wf, ((0, C2P - C2), (0, 0)))

    s_idx = jnp.arange(T_SRC)
    mask = ((s_idx >= PAD) & (s_idx < PAD + T_IN)).astype(jnp.float32)
    mask_sh = jnp.stack([mask[k:k + T_LANE] for k in range(KW)], 0)
    cvec = jnp.sum(w2 * c1[None, :, None], axis=1)
    bias = (a2[:, None] * jnp.einsum('ok,kt->ot', cvec, mask_sh, precision=hp)
            + c2[:, None])
    bias = jnp.pad(bias, ((0, C2P - C2), (0, 0)))

    tt = jnp.arange(T_LANE)[:, None]
    qq = jnp.arange(QP)[None, :]
    pmat = ((tt >= POOL_S * qq) & (tt < POOL_S * qq + POOL_W)
            & (tt < T_OUT) & (qq < N_POOL)).astype(jnp.float32) / POOL_W

    oo = jnp.arange(C2P)[:, None]
    padfix = ((oo >= C2) | (qq >= N_POOL)).astype(jnp.float32)

    wc = p['cls_w'].reshape(NCLS, C2, N_POOL)
    wc = jnp.pad(wc, ((0, 0), (0, C2P - C2), (0, QP - N_POOL)))

    wc2 = wc.transpose(2, 0, 1).reshape(QP, NCLS * C2P)
    m24 = jnp.concatenate([jnp.eye(C2P, dtype=jnp.float32)] * NCLS, axis=1)
    mtile = jnp.concatenate([m24] * (2 * CH_P), axis=0)
    gmat = jnp.kron(jnp.eye(NCLS, dtype=jnp.float32),
                    jnp.ones((C2P, 1), jnp.float32))

    bc = jnp.pad(p['cls_b'][None, :], ((0, 0), (0, QP - NCLS)))

    wf_r = wf.reshape(C2P, KW, CIN)
    w2a = jnp.pad(wf_r, ((0, 0), (0, 0), (0, CIN))).reshape(C2P, KIM2)
    w2b = jnp.pad(wf_r, ((0, 0), (0, 0), (CIN, 0))).reshape(C2P, KIM2)
    wpair = jnp.concatenate([w2a, w2b], axis=0)
    bias2 = jnp.concatenate([bias, bias], axis=0)
    return wpair, bias2, pmat, padfix, wc2, mtile, gmat, bc


def _kernel_body(x_ref, wf_ref, bias_ref, pmat_ref, padfix_ref,
                 wc2_ref, mtile_ref, gmat_ref, bc_ref, out_ref,
                 xp_s, xi_s, sq_s):
    xp_s[:, :, 0:PAD] = jnp.zeros((P_BLK, 2 * CIN, PAD), jnp.float32)
    xcat = jnp.concatenate([x_ref[0:P_BLK], x_ref[P_BLK:2 * P_BLK]], axis=1)
    xp_s[:, :, PAD:PAD + T_IN] = xcat
    xp_s[:, :, PAD + T_IN:T_SRC] = jnp.zeros(
        (P_BLK, 2 * CIN, T_SRC - PAD - T_IN), jnp.float32)

    wf = wf_ref[...]
    bias = bias_ref[...]
    out_ref[...] = jnp.zeros_like(out_ref)

    for c in range(P_BLK // CH_P):
        p0 = c * CH_P
        for k in range(KW):
            xi_s[p0:p0 + CH_P, 2 * CIN * k:2 * CIN * (k + 1), :] = \
                xp_s[p0:p0 + CH_P, :, k:k + T_LANE]

        for b in range(p0, p0 + CH_P):
            z = jnp.dot(wf, xi_s[b],
                        preferred_element_type=jnp.float32) + bias
            sq_s[b] = z * z

        sq_flat = sq_s[p0:p0 + CH_P].reshape(CH_P * C2P2, T_LANE)
        pooled = jnp.dot(sq_flat, pmat_ref[...],
                         preferred_element_type=jnp.float32)

        padfix3 = jnp.broadcast_to(padfix_ref[...], (2 * CH_P, C2P, QP))
        logp = jnp.log(pooled + padfix3.reshape(CH_P * C2P2, QP))

        u = jnp.dot(logp, wc2_ref[...],
                    preferred_element_type=jnp.float32)
        um = u * mtile_ref[...]
        v = jnp.sum(um.reshape(2 * CH_P, C2P, NCLS * C2P), axis=1)
        s = jnp.dot(v, gmat_ref[...],
                    preferred_element_type=jnp.float32)
        out_ref[2 * p0:2 * (p0 + CH_P), 0:NCLS] = s + bc_ref[:, 0:NCLS]


def kernel(x, conv1_w, conv1_b, bn1_gamma, bn1_beta, bn1_mean, bn1_var,
           conv2_w, conv2_b, bn2_gamma, bn2_beta, bn2_mean, bn2_var,
           cls_w, cls_b):
    p = dict(
        conv1_w=conv1_w, conv1_b=conv1_b,
        bn1_gamma=bn1_gamma, bn1_beta=bn1_beta,
        bn1_mean=bn1_mean, bn1_var=bn1_var,
        conv2_w=conv2_w, conv2_b=conv2_b,
        bn2_gamma=bn2_gamma, bn2_beta=bn2_beta,
        bn2_mean=bn2_mean, bn2_var=bn2_var,
        cls_w=cls_w, cls_b=cls_b,
    )
    wf, bias, pmat, padfix, wc2, mtile, gmat, bc = _fold_params(p)

    B = x.shape[0]
    x2 = x[:, 0, :, :]

    out = pl.pallas_call(
        _kernel_body,
        out_shape=jax.ShapeDtypeStruct((B, QP), jnp.float32),
        grid=(B // B_BLK,),
        in_specs=[
            pl.BlockSpec((B_BLK, CIN, T_IN), lambda g: (g, 0, 0)),
            pl.BlockSpec((C2P2, KIM2), lambda g: (0, 0)),
            pl.BlockSpec((C2P2, T_LANE), lambda g: (0, 0)),
            pl.BlockSpec((T_LANE, QP), lambda g: (0, 0)),
            pl.BlockSpec((C2P, QP), lambda g: (0, 0)),
            pl.BlockSpec((QP, NCLS * C2P), lambda g: (0, 0)),
            pl.BlockSpec((2 * CH_P * C2P, NCLS * C2P), lambda g: (0, 0)),
            pl.BlockSpec((NCLS * C2P, NCLS), lambda g: (0, 0)),
            pl.BlockSpec((1, QP), lambda g: (0, 0)),
        ],
        out_specs=pl.BlockSpec((B_BLK, QP), lambda g: (g, 0)),
        scratch_shapes=[
            pltpu.VMEM((P_BLK, 2 * CIN, T_SRC), jnp.float32),
            pltpu.VMEM((P_BLK, KIM2, T_LANE), jnp.float32),
            pltpu.VMEM((P_BLK, C2P2, T_LANE), jnp.float32),
        ],
        compiler_params=pltpu.CompilerParams(
            dimension_semantics=("parallel",),
            vmem_limit_bytes=64 * 1024 * 1024),
    )(x2, wf, bias, pmat, padfix, wc2, mtile, gmat, bc)
    outs = out[:, :NCLS].reshape(B // B_BLK, P_BLK, 2, NCLS)
    return outs.swapaxes(1, 2).reshape(B, NCLS)
